# Initial kernel scaffold; baseline (speedup 1.0000x reference)
import jax
import jax.numpy as jnp
from jax.experimental import pallas as pl


def kernel(x, edge_index_0, edge_index_1, edge_index_2, edge_index_3, edge_index_4, edge_index_5, edge_index_6, W, b):
    raise NotImplementedError("write your pallas kernel here")


# speedup vs baseline: 5.4641x; 5.4641x over previous
import functools

import jax
import jax.numpy as jnp
from jax import lax
from jax.experimental import pallas as pl
from jax.experimental.pallas import tpu as pltpu
from jax.experimental.pallas import tpu_sc as plsc

N = 10000
D = 128
T = 7
E = 45714

TILES = 16
CHUNK_A = 96
CHUNK_D = 64
EPT = 2880
N_CH_A = EPT // CHUNK_A
N_CH_D = EPT // CHUNK_D
EPAD = TILES * EPT
PAD = EPAD - E
NPAD = 10112
RPT = NPAD // TILES
T6A = N_CH_A // 2
T6D = N_CH_D // 2

_mesh = plsc.VectorSubcoreMesh(core_axis_name="c", subcore_axis_name="s",
                               num_cores=2, num_subcores=TILES)


def _phase_params(c, phase, n_ch_full, t6_split):
    if phase < 3:
        t_idx = 3 * c + phase
        return t_idx, t_idx, 0 * c, n_ch_full
    t_idx = jnp.int32(6)
    slot = 6 + c
    ch_lo = c * t6_split
    n_ch = jnp.where(c == 0, t6_split, n_ch_full - t6_split)
    return t_idx, slot, ch_lo, n_ch


@functools.partial(
    pl.kernel,
    out_type=jax.ShapeDtypeStruct((8, NPAD, D), jnp.float32),
    mesh=_mesh,
    scratch_types=[
        pltpu.VMEM((CHUNK_A,), jnp.int32),
        pltpu.VMEM((CHUNK_A,), jnp.int32),
        pltpu.VMEM((CHUNK_A, D), jnp.float32),
        pltpu.VMEM_SHARED((NPAD, D), jnp.float32),
        pltpu.SemaphoreType.DMA,
    ],
)
def _sc_aggregate(x_hbm, ei_hbm, agg_out,
                  src_v, dst_v, rows_v, agg_sh, gsem):
    c = lax.axis_index("c")
    s = lax.axis_index("s")

    def _zero_rows(i, carry):
        z16 = jnp.zeros((16,), jnp.float32)
        for j in range(D // 16):
            rows_v[i, pl.ds(j * 16, 16)] = z16
        return carry

    tile_base = s * EPT
    row_base = s * RPT

    for phase in range(4):
        t_idx, slot, ch_lo, n_ch = _phase_params(c, phase, N_CH_A, T6A)

        lax.fori_loop(0, CHUNK_A, _zero_rows, 0)
        off = 0
        while off < RPT:
            rr = min(CHUNK_A, RPT - off)
            pltpu.sync_copy(rows_v.at[pl.ds(0, rr), :],
                            agg_sh.at[pl.ds(row_base + off, rr), :])
            off += rr
        plsc.subcore_barrier()

        def _chunk(ci, carry):
            eoff = tile_base + (ch_lo + ci) * CHUNK_A
            pltpu.sync_copy(
                ei_hbm.at[pl.ds((t_idx * 2) * EPAD + eoff, CHUNK_A)], src_v)
            pltpu.sync_copy(
                ei_hbm.at[pl.ds((t_idx * 2 + 1) * EPAD + eoff, CHUNK_A)], dst_v)
            pltpu.async_copy(x_hbm.at[src_v], rows_v, gsem).wait()
            pltpu.sync_copy(rows_v, agg_sh.at[dst_v], add=True)
            return carry

        lax.fori_loop(0, n_ch, _chunk, 0)
        plsc.subcore_barrier()

        off = 0
        while off < RPT:
            rr = min(CHUNK_A, RPT - off)
            pltpu.sync_copy(agg_sh.at[pl.ds(row_base + off, rr), :],
                            rows_v.at[pl.ds(0, rr), :])
            pltpu.sync_copy(rows_v.at[pl.ds(0, rr), :],
                            agg_out.at[slot, pl.ds(row_base + off, rr), :])
            off += rr


@functools.partial(
    pl.kernel,
    out_type=jax.ShapeDtypeStruct((2, NPAD, D), jnp.float32),
    mesh=_mesh,
    scratch_types=[
        pltpu.VMEM((CHUNK_D,), jnp.int32),
        pltpu.VMEM((CHUNK_D, D), jnp.float32),
        pltpu.VMEM((CHUNK_D, D), jnp.float32),
        pltpu.VMEM_SHARED((NPAD, D), jnp.float32),
    ],
)
def _sc_degrees(ei_hbm, ones_hbm, deg_out,
                dst_v, ones_v, stage_v, deg_sh):
    c = lax.axis_index("c")
    s = lax.axis_index("s")

    def _zero_rows(i, carry):
        z16 = jnp.zeros((16,), jnp.float32)
        for j in range(D // 16):
            stage_v[i, pl.ds(j * 16, 16)] = z16
        return carry

    lax.fori_loop(0, CHUNK_D, _zero_rows, 0)

    tile_base = s * EPT
    row_base = s * RPT

    off = 0
    while off < RPT:
        rr = min(CHUNK_D, RPT - off)
        pltpu.sync_copy(stage_v.at[pl.ds(0, rr), :],
                        deg_sh.at[pl.ds(row_base + off, rr), :])
        off += rr
    plsc.subcore_barrier()

    for phase in range(4):
        t_idx, _, ch_lo, n_ch = _phase_params(c, phase, N_CH_D, T6D)
        pltpu.sync_copy(ones_hbm.at[t_idx], ones_v)

        def _chunk(ci, carry):
            eoff = tile_base + (ch_lo + ci) * CHUNK_D
            pltpu.sync_copy(
                ei_hbm.at[pl.ds((t_idx * 2 + 1) * EPAD + eoff, CHUNK_D)], dst_v)
            pltpu.sync_copy(ones_v, deg_sh.at[dst_v], add=True)
            return carry

        lax.fori_loop(0, n_ch, _chunk, 0)

    plsc.subcore_barrier()
    off = 0
    while off < RPT:
        rr = min(CHUNK_D, RPT - off)
        pltpu.sync_copy(deg_sh.at[pl.ds(row_base + off, rr), :],
                        stage_v.at[pl.ds(0, rr), :])
        pltpu.sync_copy(stage_v.at[pl.ds(0, rr), :],
                        deg_out.at[c, pl.ds(row_base + off, rr), :])
        off += rr


ROWS_BLK = 400


def _tc_body(agg_ref, deg_ref, w_ref, b_ref, out_ref):
    deg = deg_ref[0] + deg_ref[1]
    acc = jnp.zeros((ROWS_BLK, D), jnp.float32)
    for t in range(T):
        if t == 6:
            a = agg_ref[6] + agg_ref[7]
        else:
            a = agg_ref[t]
        dg = deg[:, 16 * t]
        h = a / jnp.maximum(dg, 1.0)[:, None]
        acc += jnp.dot(h, w_ref[t], preferred_element_type=jnp.float32)
    bias = jnp.sum(b_ref[...], axis=0)
    out_ref[...] = (acc + bias) * (1.0 / T)


def _tc_combine(agg, deg, W, b):
    return pl.pallas_call(
        _tc_body,
        grid=(N // ROWS_BLK,),
        in_specs=[
            pl.BlockSpec((8, ROWS_BLK, D), lambda i: (0, i, 0)),
            pl.BlockSpec((2, ROWS_BLK, D), lambda i: (0, i, 0)),
            pl.BlockSpec((T, D, D), lambda i: (0, 0, 0)),
            pl.BlockSpec((T, D), lambda i: (0, 0)),
        ],
        out_specs=pl.BlockSpec((ROWS_BLK, D), lambda i: (i, 0)),
        out_shape=jax.ShapeDtypeStruct((N, D), jnp.float32),
    )(agg, deg, W, b)


def kernel(x, edge_index_0, edge_index_1, edge_index_2, edge_index_3,
           edge_index_4, edge_index_5, edge_index_6, W, b):
    ei = jnp.stack([edge_index_0, edge_index_1, edge_index_2, edge_index_3,
                    edge_index_4, edge_index_5, edge_index_6], axis=0)
    ar = jnp.arange(PAD, dtype=jnp.int32)
    pad_src = (ar * 997) % N
    pad_dst = N + (ar % (NPAD - N))
    pad = jnp.stack([pad_src, pad_dst], axis=0)[None]
    ei_pad = jnp.concatenate(
        [ei, jnp.broadcast_to(pad, (T, 2, PAD))], axis=2).reshape(-1)
    lane_t = jnp.arange(D, dtype=jnp.int32) // 16
    ones_lanes = (lane_t[None, :] == jnp.arange(T, dtype=jnp.int32)[:, None])
    ones_const = jnp.broadcast_to(
        ones_lanes[:, None, :], (T, CHUNK_D, D)).astype(jnp.float32)
    agg = _sc_aggregate(x, ei_pad)
    deg = _sc_degrees(ei_pad, ones_const)
    return _tc_combine(agg, deg, W, b)


# speedup vs baseline: 6.9926x; 1.2797x over previous
import functools

import jax
import jax.numpy as jnp
from jax import lax
from jax.experimental import pallas as pl
from jax.experimental.pallas import tpu as pltpu
from jax.experimental.pallas import tpu_sc as plsc

N = 10000
D = 128
T = 7
E = 45714

TILES = 16
CHUNK_A = 96
CHUNK_D = 64
EPT = 2880
N_CH_A = EPT // CHUNK_A
N_CH_D = EPT // CHUNK_D
EPAD = TILES * EPT
PAD = EPAD - E
NPAD = 10112
RPT = NPAD // TILES
T6A = N_CH_A // 2
T6D = N_CH_D // 2

_mesh = plsc.VectorSubcoreMesh(core_axis_name="c", subcore_axis_name="s",
                               num_cores=2, num_subcores=TILES)


def _phase_params(c, phase, n_ch_full, t6_split):
    if phase < 3:
        t_idx = 3 * c + phase
        return t_idx, t_idx, 0 * c, n_ch_full
    t_idx = jnp.int32(6)
    slot = 6 + c
    ch_lo = c * t6_split
    n_ch = jnp.where(c == 0, t6_split, n_ch_full - t6_split)
    return t_idx, slot, ch_lo, n_ch


@functools.partial(
    pl.kernel,
    out_type=jax.ShapeDtypeStruct((8, NPAD, D), jnp.float32),
    mesh=_mesh,
    scratch_types=[
        pltpu.VMEM((2, CHUNK_A), jnp.int32),
        pltpu.VMEM((2, CHUNK_A), jnp.int32),
        pltpu.VMEM((CHUNK_A, D), jnp.float32),
        pltpu.VMEM((CHUNK_A, D), jnp.float32),
        pltpu.VMEM_SHARED((NPAD, D), jnp.float32),
        pltpu.SemaphoreType.DMA,
        pltpu.SemaphoreType.DMA,
    ],
)
def _sc_aggregate(x_hbm, ei_hbm, agg_out,
                  src_v, dst_v, rows_a, rows_b, agg_sh, gsem_a, gsem_b):
    c = lax.axis_index("c")
    s = lax.axis_index("s")

    def _zero_rows(i, carry):
        z16 = jnp.zeros((16,), jnp.float32)
        for j in range(D // 16):
            rows_a[i, pl.ds(j * 16, 16)] = z16
        return carry

    tile_base = s * EPT
    row_base = s * RPT

    for phase in range(4):
        t_idx, slot, ch_lo, n_ch = _phase_params(c, phase, N_CH_A, T6A)
        src_base = (t_idx * 2) * EPAD + tile_base + ch_lo * CHUNK_A
        dst_base = (t_idx * 2 + 1) * EPAD + tile_base + ch_lo * CHUNK_A

        lax.fori_loop(0, CHUNK_A, _zero_rows, 0)
        off = 0
        while off < RPT:
            rr = min(CHUNK_A, RPT - off)
            pltpu.sync_copy(rows_a.at[pl.ds(0, rr), :],
                            agg_sh.at[pl.ds(row_base + off, rr), :])
            off += rr
        plsc.subcore_barrier()

        def _load_idx(k, ci):
            pltpu.sync_copy(ei_hbm.at[pl.ds(src_base + ci * CHUNK_A, CHUNK_A)],
                            src_v.at[k])
            pltpu.sync_copy(ei_hbm.at[pl.ds(dst_base + ci * CHUNK_A, CHUNK_A)],
                            dst_v.at[k])

        _load_idx(0, 0)
        pltpu.async_copy(x_hbm.at[src_v.at[0]], rows_a, gsem_a)

        def _pair(j, carry):
            for k, rows_k, gsem_k, gsem_n in ((0, rows_a, gsem_a, gsem_b),
                                              (1, rows_b, gsem_b, gsem_a)):
                ci = 2 * j + k
                rows_n = rows_b if k == 0 else rows_a

                @pl.when(ci < n_ch)
                def _():
                    @pl.when(ci + 1 < n_ch)
                    def _():
                        _load_idx(1 - k, ci + 1)
                    pltpu.make_async_copy(x_hbm.at[src_v.at[k]],
                                          rows_k, gsem_k).wait()

                    @pl.when(ci + 1 < n_ch)
                    def _():
                        pltpu.async_copy(x_hbm.at[src_v.at[1 - k]],
                                         rows_n, gsem_n)
                    pltpu.sync_copy(rows_k, agg_sh.at[dst_v.at[k]], add=True)
            return carry

        lax.fori_loop(0, (n_ch + 1) // 2, _pair, 0)
        plsc.subcore_barrier()

        off = 0
        while off < RPT:
            rr = min(CHUNK_A, RPT - off)
            pltpu.sync_copy(agg_sh.at[pl.ds(row_base + off, rr), :],
                            rows_a.at[pl.ds(0, rr), :])
            pltpu.sync_copy(rows_a.at[pl.ds(0, rr), :],
                            agg_out.at[slot, pl.ds(row_base + off, rr), :])
            off += rr


@functools.partial(
    pl.kernel,
    out_type=jax.ShapeDtypeStruct((2, NPAD, D), jnp.float32),
    mesh=_mesh,
    scratch_types=[
        pltpu.VMEM((CHUNK_D,), jnp.int32),
        pltpu.VMEM((CHUNK_D, D), jnp.float32),
        pltpu.VMEM((CHUNK_D, D), jnp.float32),
        pltpu.VMEM_SHARED((NPAD, D), jnp.float32),
    ],
)
def _sc_degrees(ei_hbm, ones_hbm, deg_out,
                dst_v, ones_v, stage_v, deg_sh):
    c = lax.axis_index("c")
    s = lax.axis_index("s")

    def _zero_rows(i, carry):
        z16 = jnp.zeros((16,), jnp.float32)
        for j in range(D // 16):
            stage_v[i, pl.ds(j * 16, 16)] = z16
        return carry

    lax.fori_loop(0, CHUNK_D, _zero_rows, 0)

    tile_base = s * EPT
    row_base = s * RPT

    off = 0
    while off < RPT:
        rr = min(CHUNK_D, RPT - off)
        pltpu.sync_copy(stage_v.at[pl.ds(0, rr), :],
                        deg_sh.at[pl.ds(row_base + off, rr), :])
        off += rr
    plsc.subcore_barrier()

    for phase in range(4):
        t_idx, _, ch_lo, n_ch = _phase_params(c, phase, N_CH_D, T6D)
        pltpu.sync_copy(ones_hbm.at[t_idx], ones_v)

        def _chunk(ci, carry):
            eoff = tile_base + (ch_lo + ci) * CHUNK_D
            pltpu.sync_copy(
                ei_hbm.at[pl.ds((t_idx * 2 + 1) * EPAD + eoff, CHUNK_D)], dst_v)
            pltpu.sync_copy(ones_v, deg_sh.at[dst_v], add=True)
            return carry

        lax.fori_loop(0, n_ch, _chunk, 0)

    plsc.subcore_barrier()
    off = 0
    while off < RPT:
        rr = min(CHUNK_D, RPT - off)
        pltpu.sync_copy(deg_sh.at[pl.ds(row_base + off, rr), :],
                        stage_v.at[pl.ds(0, rr), :])
        pltpu.sync_copy(stage_v.at[pl.ds(0, rr), :],
                        deg_out.at[c, pl.ds(row_base + off, rr), :])
        off += rr


ROWS_BLK = 400


def _tc_body(agg_ref, deg_ref, w_ref, b_ref, out_ref):
    deg = deg_ref[0] + deg_ref[1]
    acc = jnp.zeros((ROWS_BLK, D), jnp.float32)
    for t in range(T):
        if t == 6:
            a = agg_ref[6] + agg_ref[7]
        else:
            a = agg_ref[t]
        dg = deg[:, 16 * t]
        h = a / jnp.maximum(dg, 1.0)[:, None]
        acc += jnp.dot(h, w_ref[t], preferred_element_type=jnp.float32)
    bias = jnp.sum(b_ref[...], axis=0)
    out_ref[...] = (acc + bias) * (1.0 / T)


def _tc_combine(agg, deg, W, b):
    return pl.pallas_call(
        _tc_body,
        grid=(N // ROWS_BLK,),
        in_specs=[
            pl.BlockSpec((8, ROWS_BLK, D), lambda i: (0, i, 0)),
            pl.BlockSpec((2, ROWS_BLK, D), lambda i: (0, i, 0)),
            pl.BlockSpec((T, D, D), lambda i: (0, 0, 0)),
            pl.BlockSpec((T, D), lambda i: (0, 0)),
        ],
        out_specs=pl.BlockSpec((ROWS_BLK, D), lambda i: (i, 0)),
        out_shape=jax.ShapeDtypeStruct((N, D), jnp.float32),
    )(agg, deg, W, b)


def kernel(x, edge_index_0, edge_index_1, edge_index_2, edge_index_3,
           edge_index_4, edge_index_5, edge_index_6, W, b):
    ei = jnp.stack([edge_index_0, edge_index_1, edge_index_2, edge_index_3,
                    edge_index_4, edge_index_5, edge_index_6], axis=0)
    ar = jnp.arange(PAD, dtype=jnp.int32)
    pad_src = (ar * 997) % N
    pad_dst = N + (ar % (NPAD - N))
    pad = jnp.stack([pad_src, pad_dst], axis=0)[None]
    ei_pad = jnp.concatenate(
        [ei, jnp.broadcast_to(pad, (T, 2, PAD))], axis=2).reshape(-1)
    lane_t = jnp.arange(D, dtype=jnp.int32) // 16
    ones_lanes = (lane_t[None, :] == jnp.arange(T, dtype=jnp.int32)[:, None])
    ones_const = jnp.broadcast_to(
        ones_lanes[:, None, :], (T, CHUNK_D, D)).astype(jnp.float32)
    agg = _sc_aggregate(x, ei_pad)
    deg = _sc_degrees(ei_pad, ones_const)
    return _tc_combine(agg, deg, W, b)


# speedup vs baseline: 9.8961x; 1.4152x over previous
import functools

import jax
import jax.numpy as jnp
from jax import lax
from jax.experimental import pallas as pl
from jax.experimental.pallas import tpu as pltpu
from jax.experimental.pallas import tpu_sc as plsc

N = 10000
D = 128
T = 7
E = 45714

TILES = 16
CHUNK_A = 96
CHUNK_D = 64
EPT = 2880
N_CH_A = EPT // CHUNK_A
N_CH_D = EPT // CHUNK_D
EPAD = TILES * EPT
PAD = EPAD - E
NPAD = 10112
RPT = NPAD // TILES
T6A = N_CH_A // 2
T6D = N_CH_D // 2

_mesh = plsc.VectorSubcoreMesh(core_axis_name="c", subcore_axis_name="s",
                               num_cores=2, num_subcores=TILES)


def _phase_params(c, phase, n_ch_full, t6_split):
    if phase < 3:
        t_idx = 3 * c + phase
        return t_idx, t_idx, 0 * c, n_ch_full
    t_idx = jnp.int32(6)
    slot = 6 + c
    ch_lo = c * t6_split
    n_ch = jnp.where(c == 0, t6_split, n_ch_full - t6_split)
    return t_idx, slot, ch_lo, n_ch


@functools.partial(
    pl.kernel,
    out_type=jax.ShapeDtypeStruct((8, NPAD, D), jnp.float32),
    mesh=_mesh,
    scratch_types=[
        pltpu.VMEM((2, CHUNK_A), jnp.int32),
        pltpu.VMEM((2, CHUNK_A), jnp.int32),
        pltpu.VMEM((CHUNK_A, D), jnp.float32),
        pltpu.VMEM((CHUNK_A, D), jnp.float32),
        pltpu.VMEM_SHARED((NPAD, D), jnp.float32),
        pltpu.SemaphoreType.DMA,
        pltpu.SemaphoreType.DMA,
    ],
)
def _sc_aggregate(x_hbm, ei_hbm, agg_out,
                  src_v, dst_v, rows_a, rows_b, agg_sh, gsem_a, gsem_b):
    c = lax.axis_index("c")
    s = lax.axis_index("s")

    def _zero_rows(i, carry):
        z16 = jnp.zeros((16,), jnp.float32)
        for j in range(D // 16):
            rows_a[i, pl.ds(j * 16, 16)] = z16
        return carry

    tile_base = s * EPT
    row_base = s * RPT

    for phase in range(4):
        t_idx, slot, ch_lo, n_ch = _phase_params(c, phase, N_CH_A, T6A)
        src_base = (t_idx * 2) * EPAD + tile_base + ch_lo * CHUNK_A
        dst_base = (t_idx * 2 + 1) * EPAD + tile_base + ch_lo * CHUNK_A

        lax.fori_loop(0, CHUNK_A, _zero_rows, 0)
        off = 0
        while off < RPT:
            rr = min(CHUNK_A, RPT - off)
            pltpu.sync_copy(rows_a.at[pl.ds(0, rr), :],
                            agg_sh.at[pl.ds(row_base + off, rr), :])
            off += rr
        plsc.subcore_barrier()

        def _load_idx(k, ci):
            pltpu.sync_copy(ei_hbm.at[pl.ds(src_base + ci * CHUNK_A, CHUNK_A)],
                            src_v.at[k])
            pltpu.sync_copy(ei_hbm.at[pl.ds(dst_base + ci * CHUNK_A, CHUNK_A)],
                            dst_v.at[k])

        _load_idx(0, 0)
        pltpu.async_copy(x_hbm.at[src_v.at[0]], rows_a, gsem_a)

        def _pair(j, carry):
            for k, rows_k, gsem_k, gsem_n in ((0, rows_a, gsem_a, gsem_b),
                                              (1, rows_b, gsem_b, gsem_a)):
                ci = 2 * j + k
                rows_n = rows_b if k == 0 else rows_a

                @pl.when(ci < n_ch)
                def _():
                    @pl.when(ci + 1 < n_ch)
                    def _():
                        _load_idx(1 - k, ci + 1)
                    pltpu.make_async_copy(x_hbm.at[src_v.at[k]],
                                          rows_k, gsem_k).wait()

                    @pl.when(ci + 1 < n_ch)
                    def _():
                        pltpu.async_copy(x_hbm.at[src_v.at[1 - k]],
                                         rows_n, gsem_n)
                    pltpu.sync_copy(rows_k, agg_sh.at[dst_v.at[k]], add=True)
            return carry

        lax.fori_loop(0, (n_ch + 1) // 2, _pair, 0)
        plsc.subcore_barrier()

        off = 0
        while off < RPT:
            rr = min(CHUNK_A, RPT - off)
            pltpu.sync_copy(agg_sh.at[pl.ds(row_base + off, rr), :],
                            rows_a.at[pl.ds(0, rr), :])
            pltpu.sync_copy(rows_a.at[pl.ds(0, rr), :],
                            agg_out.at[slot, pl.ds(row_base + off, rr), :])
            off += rr


@functools.partial(
    pl.kernel,
    out_type=jax.ShapeDtypeStruct((8 * TILES * NPAD,), jnp.float32),
    mesh=_mesh,
    scratch_types=[
        pltpu.VMEM((EPT,), jnp.int32),
        pltpu.VMEM((NPAD,), jnp.float32),
    ],
    compiler_params=pltpu.CompilerParams(needs_layout_passes=False),
)
def _sc_degrees(ei_hbm, deg_out, dst_all, hist_v):
    c = lax.axis_index("c")
    s = lax.axis_index("s")

    def _zero_hist(i, carry):
        hist_v[pl.ds(i * 16, 16)] = jnp.zeros((16,), jnp.float32)
        return carry

    lax.fori_loop(0, NPAD // 16, _zero_hist, 0)

    tile_base = s * EPT
    ones16 = jnp.ones((16,), jnp.float32)

    for phase in range(4):
        if phase < 3:
            t_idx, slot = 3 * c + phase, 3 * c + phase
            ecnt, ch_lo = EPT, 0 * c
        else:
            t_idx, slot = jnp.int32(6), 6 + c
            ecnt, ch_lo = EPT // 2, c * (EPT // 2)

        pltpu.sync_copy(
            ei_hbm.at[pl.ds((t_idx * 2 + 1) * EPAD + tile_base + ch_lo, ecnt)],
            dst_all.at[pl.ds(0, ecnt)])

        def _group(g, carry):
            idx16 = dst_all[pl.ds(g * 16, 16)]
            plsc.addupdate_scatter(hist_v, [idx16], ones16)
            return carry

        lax.fori_loop(0, ecnt // 16, _group, 0)

        pltpu.sync_copy(hist_v,
                        deg_out.at[pl.ds((slot * TILES + s) * NPAD, NPAD)])
        if phase < 3:
            lax.fori_loop(0, NPAD // 16, _zero_hist, 0)


ROWS_BLK = 400


def _tc_degred_body(deg_ref, out_ref):
    dsum = jnp.sum(deg_ref[...], axis=1)
    out_ref[...] = dsum.T


def _tc_degred(deg):
    return pl.pallas_call(
        _tc_degred_body,
        grid=(NPAD // 128,),
        in_specs=[pl.BlockSpec((8, TILES, 128), lambda i: (0, 0, i))],
        out_specs=pl.BlockSpec((128, 8), lambda i: (i, 0)),
        out_shape=jax.ShapeDtypeStruct((NPAD, 8), jnp.float32),
    )(deg)


def _tc_body(agg_ref, deg_ref, w_ref, b_ref, out_ref):
    deg = deg_ref[...]
    acc = jnp.zeros((ROWS_BLK, D), jnp.float32)
    for t in range(T):
        if t == 6:
            a = agg_ref[6] + agg_ref[7]
            dg = deg[:, 6] + deg[:, 7]
        else:
            a = agg_ref[t]
            dg = deg[:, t]
        h = a / jnp.maximum(dg, 1.0)[:, None]
        acc += jnp.dot(h, w_ref[t], preferred_element_type=jnp.float32)
    bias = jnp.sum(b_ref[...], axis=0)
    out_ref[...] = (acc + bias) * (1.0 / T)


def _tc_combine(agg, deg, W, b):
    return pl.pallas_call(
        _tc_body,
        grid=(N // ROWS_BLK,),
        in_specs=[
            pl.BlockSpec((8, ROWS_BLK, D), lambda i: (0, i, 0)),
            pl.BlockSpec((ROWS_BLK, 8), lambda i: (i, 0)),
            pl.BlockSpec((T, D, D), lambda i: (0, 0, 0)),
            pl.BlockSpec((T, D), lambda i: (0, 0)),
        ],
        out_specs=pl.BlockSpec((ROWS_BLK, D), lambda i: (i, 0)),
        out_shape=jax.ShapeDtypeStruct((N, D), jnp.float32),
    )(agg, deg, W, b)


def kernel(x, edge_index_0, edge_index_1, edge_index_2, edge_index_3,
           edge_index_4, edge_index_5, edge_index_6, W, b):
    ei = jnp.stack([edge_index_0, edge_index_1, edge_index_2, edge_index_3,
                    edge_index_4, edge_index_5, edge_index_6], axis=0)
    ar = jnp.arange(PAD, dtype=jnp.int32)
    pad_src = (ar * 997) % N
    pad_dst = N + (ar % (NPAD - N))
    pad = jnp.stack([pad_src, pad_dst], axis=0)[None]
    ei_pad = jnp.concatenate(
        [ei, jnp.broadcast_to(pad, (T, 2, PAD))], axis=2).reshape(-1)
    agg = _sc_aggregate(x, ei_pad)
    deg = _tc_degred(_sc_degrees(ei_pad).reshape(8, TILES, NPAD))
    return _tc_combine(agg, deg, W, b)


# speedup vs baseline: 9.9839x; 1.0089x over previous
import functools

import jax
import jax.numpy as jnp
from jax import lax
from jax.experimental import pallas as pl
from jax.experimental.pallas import tpu as pltpu
from jax.experimental.pallas import tpu_sc as plsc

N = 10000
D = 128
T = 7
E = 45714

TILES = 16
CHUNK_A = 96
CHUNK_D = 64
EPT = 2880
N_CH_A = EPT // CHUNK_A
N_CH_D = EPT // CHUNK_D
EPAD = TILES * EPT
PAD = EPAD - E
NPAD = 10112
RPT = NPAD // TILES
T6A = N_CH_A // 2
T6D = N_CH_D // 2

_mesh = plsc.VectorSubcoreMesh(core_axis_name="c", subcore_axis_name="s",
                               num_cores=2, num_subcores=TILES)


def _phase_params(c, phase, n_ch_full, t6_split):
    if phase < 3:
        t_idx = 3 * c + phase
        return t_idx, t_idx, 0 * c, n_ch_full
    t_idx = jnp.int32(6)
    slot = 6 + c
    ch_lo = c * t6_split
    n_ch = jnp.where(c == 0, t6_split, n_ch_full - t6_split)
    return t_idx, slot, ch_lo, n_ch


@functools.partial(
    pl.kernel,
    out_type=(
        jax.ShapeDtypeStruct((8, NPAD, D), jnp.float32),
        jax.ShapeDtypeStruct((8 * TILES * NPAD,), jnp.float32),
    ),
    mesh=_mesh,
    scratch_types=[
        pltpu.VMEM((EPT,), jnp.int32),
        pltpu.VMEM((N_CH_A, CHUNK_A), jnp.int32),
        pltpu.VMEM((NPAD,), jnp.float32),
        pltpu.VMEM((CHUNK_A, D), jnp.float32),
        pltpu.VMEM((CHUNK_A, D), jnp.float32),
        pltpu.VMEM_SHARED((NPAD, D), jnp.float32),
        pltpu.SemaphoreType.DMA,
        pltpu.SemaphoreType.DMA,
    ],
    compiler_params=pltpu.CompilerParams(needs_layout_passes=False),
)
def _sc_aggregate(x_hbm, ei_hbm, ds_hbm, agg_out, deg_out,
                  src_all, dst2d, hist_v, rows_a, rows_b, agg_sh,
                  gsem_a, gsem_b):
    c = lax.axis_index("c")
    s = lax.axis_index("s")

    def _zero_hist(i, carry):
        hist_v[pl.ds(i * 16, 16)] = jnp.zeros((16,), jnp.float32)
        return carry

    lax.fori_loop(0, NPAD // 16, _zero_hist, 0)
    ones16 = jnp.ones((16,), jnp.float32)

    def _zero_rows(i, carry):
        z16 = jnp.zeros((16,), jnp.float32)
        for j in range(D // 16):
            rows_a[i, pl.ds(j * 16, 16)] = z16
        return carry

    tile_base = s * EPT
    row_base = s * RPT

    for phase in range(4):
        t_idx, slot, ch_lo, n_ch = _phase_params(c, phase, N_CH_A, T6A)

        pltpu.sync_copy(
            ei_hbm.at[pl.ds((t_idx * 2) * EPAD + tile_base, EPT)], src_all)
        pltpu.sync_copy(ds_hbm.at[t_idx, s], dst2d)

        lax.fori_loop(0, CHUNK_A, _zero_rows, 0)
        off = 0
        while off < RPT:
            rr = min(CHUNK_A, RPT - off)
            pltpu.sync_copy(rows_a.at[pl.ds(0, rr), :],
                            agg_sh.at[pl.ds(row_base + off, rr), :])
            off += rr
        plsc.subcore_barrier()

        def _gather(ci, rows_k, gsem_k):
            idx = src_all.at[pl.ds((ch_lo + ci) * CHUNK_A, CHUNK_A)]
            pltpu.async_copy(x_hbm.at[idx], rows_k, gsem_k)

        _gather(0, rows_a, gsem_a)

        def _pair(j, carry):
            for k, rows_k, gsem_k, gsem_n in ((0, rows_a, gsem_a, gsem_b),
                                              (1, rows_b, gsem_b, gsem_a)):
                ci = 2 * j + k
                rows_n = rows_b if k == 0 else rows_a

                @pl.when(ci < n_ch)
                def _():
                    @pl.when(ci + 1 < n_ch)
                    def _():
                        _gather(ci + 1, rows_n, gsem_n)
                    for g in range(CHUNK_A // 16):
                        idx16 = dst2d[ch_lo + ci, pl.ds(g * 16, 16)]
                        plsc.addupdate_scatter(hist_v, [idx16], ones16)
                    pltpu.make_async_copy(x_hbm.at[src_all.at[pl.ds(0, CHUNK_A)]],
                                          rows_k, gsem_k).wait()
                    pltpu.sync_copy(rows_k, agg_sh.at[dst2d.at[ch_lo + ci]],
                                    add=True)
            return carry

        lax.fori_loop(0, (n_ch + 1) // 2, _pair, 0)
        plsc.subcore_barrier()

        pltpu.sync_copy(hist_v,
                        deg_out.at[pl.ds((slot * TILES + s) * NPAD, NPAD)])
        if phase < 3:
            lax.fori_loop(0, NPAD // 16, _zero_hist, 0)
        off = 0
        while off < RPT:
            rr = min(CHUNK_A, RPT - off)
            pltpu.sync_copy(agg_sh.at[pl.ds(row_base + off, rr), :],
                            rows_a.at[pl.ds(0, rr), :])
            pltpu.sync_copy(rows_a.at[pl.ds(0, rr), :],
                            agg_out.at[slot, pl.ds(row_base + off, rr), :])
            off += rr


ROWS_BLK = 400


def _tc_degred_body(deg_ref, out_ref):
    dsum = jnp.sum(deg_ref[...], axis=1)
    out_ref[...] = dsum.T


def _tc_degred(deg):
    return pl.pallas_call(
        _tc_degred_body,
        grid=(NPAD // 128,),
        in_specs=[pl.BlockSpec((8, TILES, 128), lambda i: (0, 0, i))],
        out_specs=pl.BlockSpec((128, 8), lambda i: (i, 0)),
        out_shape=jax.ShapeDtypeStruct((NPAD, 8), jnp.float32),
    )(deg)


def _tc_body(agg_ref, deg_ref, w_ref, b_ref, out_ref):
    deg = deg_ref[...]
    acc = jnp.zeros((ROWS_BLK, D), jnp.float32)
    for t in range(T):
        if t == 6:
            a = agg_ref[6] + agg_ref[7]
            dg = deg[:, 6] + deg[:, 7]
        else:
            a = agg_ref[t]
            dg = deg[:, t]
        h = a / jnp.maximum(dg, 1.0)[:, None]
        acc += jnp.dot(h, w_ref[t], preferred_element_type=jnp.float32)
    bias = jnp.sum(b_ref[...], axis=0)
    out_ref[...] = (acc + bias) * (1.0 / T)


def _tc_combine(agg, deg, W, b):
    return pl.pallas_call(
        _tc_body,
        grid=(N // ROWS_BLK,),
        in_specs=[
            pl.BlockSpec((8, ROWS_BLK, D), lambda i: (0, i, 0)),
            pl.BlockSpec((ROWS_BLK, 8), lambda i: (i, 0)),
            pl.BlockSpec((T, D, D), lambda i: (0, 0, 0)),
            pl.BlockSpec((T, D), lambda i: (0, 0)),
        ],
        out_specs=pl.BlockSpec((ROWS_BLK, D), lambda i: (i, 0)),
        out_shape=jax.ShapeDtypeStruct((N, D), jnp.float32),
    )(agg, deg, W, b)


def kernel(x, edge_index_0, edge_index_1, edge_index_2, edge_index_3,
           edge_index_4, edge_index_5, edge_index_6, W, b):
    ei = jnp.stack([edge_index_0, edge_index_1, edge_index_2, edge_index_3,
                    edge_index_4, edge_index_5, edge_index_6], axis=0)
    ar = jnp.arange(PAD, dtype=jnp.int32)
    pad_src = (ar * 997) % N
    pad_dst = N + (ar % (NPAD - N))
    pad = jnp.stack([pad_src, pad_dst], axis=0)[None]
    ei_pad = jnp.concatenate(
        [ei, jnp.broadcast_to(pad, (T, 2, PAD))], axis=2).reshape(-1)
    ds_resh = ei_pad.reshape(T, 2, TILES, N_CH_A, CHUNK_A)[:, 1]
    agg, deg = _sc_aggregate(x, ei_pad, ds_resh)
    deg = _tc_degred(deg.reshape(8, TILES, NPAD))
    return _tc_combine(agg, deg, W, b)


# speedup vs baseline: 10.0706x; 1.0087x over previous
import functools

import jax
import jax.numpy as jnp
from jax import lax
from jax.experimental import pallas as pl
from jax.experimental.pallas import tpu as pltpu
from jax.experimental.pallas import tpu_sc as plsc

N = 10000
D = 128
T = 7
E = 45714

TILES = 16
CHUNK_A = 96
CHUNK_D = 64
EPT = 2880
N_CH_A = EPT // CHUNK_A
N_CH_D = EPT // CHUNK_D
EPAD = TILES * EPT
PAD = EPAD - E
NPAD = 10112
RPT = NPAD // TILES
T6A = N_CH_A // 2
T6D = N_CH_D // 2

_mesh = plsc.VectorSubcoreMesh(core_axis_name="c", subcore_axis_name="s",
                               num_cores=2, num_subcores=TILES)


def _phase_params(c, phase, n_ch_full, t6_split):
    if phase < 3:
        t_idx = 3 * c + phase
        return t_idx, t_idx, 0 * c, n_ch_full
    t_idx = jnp.int32(6)
    slot = 6 + c
    ch_lo = c * t6_split
    n_ch = jnp.where(c == 0, t6_split, n_ch_full - t6_split)
    return t_idx, slot, ch_lo, n_ch


@functools.partial(
    pl.kernel,
    out_type=(
        jax.ShapeDtypeStruct((8, NPAD, D), jnp.float32),
        jax.ShapeDtypeStruct((8 * TILES * NPAD,), jnp.float32),
    ),
    mesh=_mesh,
    scratch_types=[
        pltpu.VMEM((EPT,), jnp.int32),
        pltpu.VMEM((N_CH_A, CHUNK_A), jnp.int32),
        pltpu.VMEM((NPAD,), jnp.float32),
        pltpu.VMEM((CHUNK_A, D), jnp.float32),
        pltpu.VMEM((CHUNK_A, D), jnp.float32),
        pltpu.VMEM_SHARED((NPAD, D), jnp.float32),
        pltpu.SemaphoreType.DMA,
        pltpu.SemaphoreType.DMA,
    ],
    compiler_params=pltpu.CompilerParams(needs_layout_passes=False),
)
def _sc_aggregate(x_hbm, ei_hbm, ds_hbm, agg_out, deg_out,
                  src_all, dst2d, hist_v, rows_a, rows_b, agg_sh,
                  gsem_a, gsem_b):
    c = lax.axis_index("c")
    s = lax.axis_index("s")

    def _zero_hist(i, carry):
        hist_v[pl.ds(i * 16, 16)] = jnp.zeros((16,), jnp.float32)
        return carry

    lax.fori_loop(0, NPAD // 16, _zero_hist, 0)
    ones16 = jnp.ones((16,), jnp.float32)

    def _zero_rows(i, carry):
        z16 = jnp.zeros((16,), jnp.float32)
        for j in range(D // 16):
            rows_a[i, pl.ds(j * 16, 16)] = z16
        return carry

    tile_base = s * EPT
    row_base = s * RPT

    for phase in range(4):
        t_idx, slot, ch_lo, n_ch = _phase_params(c, phase, N_CH_A, T6A)

        pltpu.sync_copy(
            ei_hbm.at[pl.ds((t_idx * 2) * EPAD + tile_base, EPT)], src_all)
        pltpu.sync_copy(ds_hbm.at[t_idx, s], dst2d)

        lax.fori_loop(0, CHUNK_A, _zero_rows, 0)
        off = 0
        while off < RPT:
            rr = min(CHUNK_A, RPT - off)
            pltpu.sync_copy(rows_a.at[pl.ds(0, rr), :],
                            agg_sh.at[pl.ds(row_base + off, rr), :])
            off += rr
        plsc.subcore_barrier()

        def _gather(ci, rows_k, gsem_k):
            idx = src_all.at[pl.ds((ch_lo + ci) * CHUNK_A, CHUNK_A)]
            pltpu.async_copy(x_hbm.at[idx], rows_k, gsem_k)

        _gather(0, rows_a, gsem_a)

        def _pair(j, carry):
            for k, rows_k, gsem_k, gsem_n in ((0, rows_a, gsem_a, gsem_b),
                                              (1, rows_b, gsem_b, gsem_a)):
                ci = 2 * j + k
                rows_n = rows_b if k == 0 else rows_a

                @pl.when(ci < n_ch)
                def _():
                    @pl.when(ci + 1 < n_ch)
                    def _():
                        _gather(ci + 1, rows_n, gsem_n)
                    for g in range(CHUNK_A // 16):
                        idx16 = dst2d[ch_lo + ci, pl.ds(g * 16, 16)]
                        plsc.addupdate_scatter(hist_v, [idx16], ones16)
                    pltpu.make_async_copy(x_hbm.at[src_all.at[pl.ds(0, CHUNK_A)]],
                                          rows_k, gsem_k).wait()
                    pltpu.sync_copy(rows_k, agg_sh.at[dst2d.at[ch_lo + ci]],
                                    add=True)
            return carry

        lax.fori_loop(0, (n_ch + 1) // 2, _pair, 0)
        plsc.subcore_barrier()

        pltpu.sync_copy(hist_v,
                        deg_out.at[pl.ds((slot * TILES + s) * NPAD, NPAD)])
        if phase < 3:
            lax.fori_loop(0, NPAD // 16, _zero_hist, 0)
        pltpu.sync_copy(agg_sh.at[pl.ds(row_base, RPT), :],
                        agg_out.at[slot, pl.ds(row_base, RPT), :])


ROWS_BLK = 400


def _tc_degred_body(deg_ref, out_ref):
    dsum = jnp.sum(deg_ref[...], axis=1)
    out_ref[...] = dsum.T


def _tc_degred(deg):
    return pl.pallas_call(
        _tc_degred_body,
        grid=(NPAD // 128,),
        in_specs=[pl.BlockSpec((8, TILES, 128), lambda i: (0, 0, i))],
        out_specs=pl.BlockSpec((128, 8), lambda i: (i, 0)),
        out_shape=jax.ShapeDtypeStruct((NPAD, 8), jnp.float32),
    )(deg)


def _tc_body(agg_ref, deg_ref, w_ref, b_ref, out_ref):
    deg = deg_ref[...]
    acc = jnp.zeros((ROWS_BLK, D), jnp.float32)
    for t in range(T):
        if t == 6:
            a = agg_ref[6] + agg_ref[7]
            dg = deg[:, 6] + deg[:, 7]
        else:
            a = agg_ref[t]
            dg = deg[:, t]
        h = a / jnp.maximum(dg, 1.0)[:, None]
        acc += jnp.dot(h, w_ref[t], preferred_element_type=jnp.float32)
    bias = jnp.sum(b_ref[...], axis=0)
    out_ref[...] = (acc + bias) * (1.0 / T)


def _tc_combine(agg, deg, W, b):
    return pl.pallas_call(
        _tc_body,
        grid=(N // ROWS_BLK,),
        in_specs=[
            pl.BlockSpec((8, ROWS_BLK, D), lambda i: (0, i, 0)),
            pl.BlockSpec((ROWS_BLK, 8), lambda i: (i, 0)),
            pl.BlockSpec((T, D, D), lambda i: (0, 0, 0)),
            pl.BlockSpec((T, D), lambda i: (0, 0)),
        ],
        out_specs=pl.BlockSpec((ROWS_BLK, D), lambda i: (i, 0)),
        out_shape=jax.ShapeDtypeStruct((N, D), jnp.float32),
    )(agg, deg, W, b)


def kernel(x, edge_index_0, edge_index_1, edge_index_2, edge_index_3,
           edge_index_4, edge_index_5, edge_index_6, W, b):
    ei = jnp.stack([edge_index_0, edge_index_1, edge_index_2, edge_index_3,
                    edge_index_4, edge_index_5, edge_index_6], axis=0)
    ar = jnp.arange(PAD, dtype=jnp.int32)
    pad_src = (ar * 997) % N
    pad_dst = N + (ar % (NPAD - N))
    pad = jnp.stack([pad_src, pad_dst], axis=0)[None]
    ei_pad = jnp.concatenate(
        [ei, jnp.broadcast_to(pad, (T, 2, PAD))], axis=2).reshape(-1)
    ds_resh = ei_pad.reshape(T, 2, TILES, N_CH_A, CHUNK_A)[:, 1]
    agg, deg = _sc_aggregate(x, ei_pad, ds_resh)
    deg = _tc_degred(deg.reshape(8, TILES, NPAD))
    return _tc_combine(agg, deg, W, b)
